# baseline (speedup 1.0000x reference)
import functools
import jax
import jax.numpy as jnp
from jax import lax
from jax.experimental import pallas as pl
from jax.experimental.pallas import tpu as pltpu
from jax.experimental.pallas import tpu_sc as plsc


def _sc_gather_rows(table, idx):
    V, D = table.shape
    M = idx.shape[0]
    NW = 32
    bpw = M // NW
    CH = min(bpw, 128)
    nch = bpw // CH
    mesh = plsc.VectorSubcoreMesh(core_axis_name="c", subcore_axis_name="s")

    def body(table_hbm, idx_hbm, out_hbm, idx_v, rows_v, sem):
        wid = lax.axis_index("s") * 2 + lax.axis_index("c")
        base = wid * bpw

        def step(c, carry):
            off = pl.multiple_of(base + c * CH, 8)
            pltpu.sync_copy(idx_hbm.at[pl.ds(off, CH)], idx_v)
            pltpu.async_copy(table_hbm.at[idx_v], rows_v, sem).wait()
            pltpu.sync_copy(rows_v, out_hbm.at[pl.ds(off, CH)])
            return carry

        lax.fori_loop(0, nch, step, 0)

    return pl.kernel(
        body,
        out_type=jax.ShapeDtypeStruct((M, D), jnp.float32),
        mesh=mesh,
        scratch_types=[pltpu.VMEM((CH,), jnp.int32),
                       pltpu.VMEM((CH, D), jnp.float32),
                       pltpu.SemaphoreType.DMA],
        compiler_params=pltpu.CompilerParams(use_tc_tiling_on_sc=False),
    )(table, idx)


def _fps_body(xyz_ref, out_ref, oxyz_ref, *, n, npoint, cols):
    X = xyz_ref[0, 0]
    Y = xyz_ref[0, 1]
    Z = xyz_ref[0, 2]
    r8 = lax.broadcasted_iota(jnp.int32, (8, cols), 0)
    c8 = lax.broadcasted_iota(jnp.int32, (8, cols), 1)
    jflat = r8 * cols + c8
    real = jflat < n
    dmin0 = jnp.where(real, jnp.full((8, cols), 1e10, jnp.float32),
                      jnp.full((8, cols), -1.0, jnp.float32))

    ocols = npoint // 8
    ro = lax.broadcasted_iota(jnp.int32, (8, ocols), 0)
    co = lax.broadcasted_iota(jnp.int32, (8, ocols), 1)
    oflat = ro * ocols + co
    inds0 = jnp.zeros((8, ocols), jnp.int32)
    zf = jnp.zeros((8, ocols), jnp.float32)

    def body(i, state):
        dmin, last, inds, nx, ny, nz = state
        sel = jflat == last
        xi = jnp.sum(jnp.where(sel, X, 0.0))
        yi = jnp.sum(jnp.where(sel, Y, 0.0))
        zi = jnp.sum(jnp.where(sel, Z, 0.0))
        prev = oflat == (i - 1)
        nx = jnp.where(prev, xi, nx)
        ny = jnp.where(prev, yi, ny)
        nz = jnp.where(prev, zi, nz)
        dx = X - xi
        dy = Y - yi
        dz = Z - zi
        d = (dx * dx + dy * dy) + dz * dz
        dmin = jnp.minimum(dmin, jnp.where(real, d, -1.0))
        m = jnp.max(dmin)
        nxt = jnp.min(jnp.where(dmin == m, jflat, jnp.int32(1 << 30)))
        inds = jnp.where(oflat == i, nxt, inds)
        return dmin, nxt, inds, nx, ny, nz

    state = lax.fori_loop(1, npoint, body,
                          (dmin0, jnp.int32(0), inds0, zf, zf, zf))
    _, last, inds, nx, ny, nz = state
    sel = jflat == last
    xi = jnp.sum(jnp.where(sel, X, 0.0))
    yi = jnp.sum(jnp.where(sel, Y, 0.0))
    zi = jnp.sum(jnp.where(sel, Z, 0.0))
    fin = oflat == (npoint - 1)
    out_ref[0] = inds
    oxyz_ref[0, 0] = jnp.where(fin, xi, nx)
    oxyz_ref[0, 1] = jnp.where(fin, yi, ny)
    oxyz_ref[0, 2] = jnp.where(fin, zi, nz)


def _fps(xyz, npoint):
    B, N, _ = xyz.shape
    cols = ((N + 7) // 8 + 127) // 128 * 128
    npad = 8 * cols
    xt = jnp.moveaxis(xyz, -1, 1)
    xt = jnp.pad(xt, ((0, 0), (0, 0), (0, npad - N)))
    xt = xt.reshape(B, 3, 8, cols)
    out, oxyz = pl.pallas_call(
        functools.partial(_fps_body, n=N, npoint=npoint, cols=cols),
        grid=(B,),
        in_specs=[pl.BlockSpec((1, 3, 8, cols), lambda b: (b, 0, 0, 0))],
        out_specs=[
            pl.BlockSpec((1, 8, npoint // 8), lambda b: (b, 0, 0)),
            pl.BlockSpec((1, 3, 8, npoint // 8), lambda b: (b, 0, 0, 0)),
        ],
        out_shape=[
            jax.ShapeDtypeStruct((B, 8, npoint // 8), jnp.int32),
            jax.ShapeDtypeStruct((B, 3, 8, npoint // 8), jnp.float32),
        ],
    )(xt)
    new_xyz = jnp.moveaxis(oxyz.reshape(B, 3, npoint), 1, 2)
    return out.reshape(B, npoint), new_xyz


def _ballq_body(cen_ref, xyzt_ref, out_ref, *, n, npad, nsample, r2, blk):
    cen = cen_ref[0]
    cx = cen[:, 0:1]
    cy = cen[:, 1:2]
    cz = cen[:, 2:3]
    X = xyzt_ref[0, 0:1, :]
    Y = xyzt_ref[0, 1:2, :]
    Z = xyzt_ref[0, 2:3, :]
    a2 = (cx * cx + cy * cy) + cz * cz
    b2 = (X * X + Y * Y) + Z * Z
    dot = lax.dot_general(cen, xyzt_ref[0], (((1,), (0,)), ((), ())),
                          precision=lax.Precision.DEFAULT,
                          preferred_element_type=jnp.float32)
    sqd = a2 + b2 - 2.0 * dot
    jj = lax.broadcasted_iota(jnp.int32, (blk, npad), 1)
    mask = (sqd < r2) & (jj < n)
    score0 = jnp.where(mask, jj, jnp.int32(n))

    ki = lax.broadcasted_iota(jnp.int32, (blk, nsample), 1)
    out0 = jnp.zeros((blk, nsample), jnp.int32)

    def body(k, state):
        prev, out = state
        score = jnp.where(jj > prev, score0, jnp.int32(n))
        jmin = jnp.min(score, axis=1, keepdims=True)
        out = jnp.where(ki == k, jmin, out)
        return jmin, out

    _, out = lax.fori_loop(0, nsample, body,
                           (jnp.full((blk, 1), -1, jnp.int32), out0))
    first = out[:, 0:1]
    out_ref[0] = jnp.where(out < n, out, first)


def _ball_query(new_xyz, xyz, radius, nsample, blk):
    import numpy as np
    B, np_, _ = new_xyz.shape
    N = xyz.shape[1]
    npad = (N + 127) // 128 * 128
    xt = jnp.moveaxis(xyz, -1, 1)
    xt = jnp.pad(xt, ((0, 0), (0, 0), (0, npad - N)), constant_values=1e6)
    r2 = float(np.float32(radius * radius))
    out = pl.pallas_call(
        functools.partial(_ballq_body, n=N, npad=npad, nsample=nsample,
                          r2=r2, blk=blk),
        grid=(B, np_ // blk),
        in_specs=[
            pl.BlockSpec((1, blk, 3), lambda b, i: (b, i, 0)),
            pl.BlockSpec((1, 3, npad), lambda b, i: (b, 0, 0)),
        ],
        out_specs=pl.BlockSpec((1, blk, nsample), lambda b, i: (b, i, 0)),
        out_shape=jax.ShapeDtypeStruct((B, np_, nsample), jnp.int32),
    )(new_xyz, xt)
    return out


def _sa_mlp_body(gx_ref, cen_ref, *w_refs, nsample, blk, radius, nfeat):
    if nfeat:
        gf_ref = w_refs[0]
        w_refs = w_refs[1:]
    out_ref = w_refs[-1]
    w_refs = w_refs[:-1]
    gx = (gx_ref[0] - cen_ref[0]) / radius
    h = None
    nw = len(w_refs) // 2
    for li in range(nw):
        W = w_refs[2 * li][...]
        b = w_refs[2 * li + 1][...]
        if li == 0:
            acc = jnp.dot(gx[:, :3], W[:3, :],
                          preferred_element_type=jnp.float32)
            if nfeat:
                acc = acc + jnp.dot(gf_ref[0], W[3:3 + nfeat, :],
                                    preferred_element_type=jnp.float32)
        else:
            acc = jnp.dot(h, W, preferred_element_type=jnp.float32)
        h = jnp.maximum(acc + b, 0.0)
    rows, cout = h.shape
    h3 = h.reshape(blk, nsample, cout)
    out_ref[0] = jnp.max(h3, axis=1)


def _sa_mlp(grouped_xyz, cen_exp, grouped_f, ws, radius, nsample, blk):
    B, rows_total, _ = grouped_xyz.shape
    np_ = rows_total // nsample
    nfeat = 0 if grouped_f is None else grouped_f.shape[-1]
    cout = ws[-1][0].shape[1]
    rows_blk = blk * nsample

    args = [grouped_xyz, cen_exp]
    in_specs = [
        pl.BlockSpec((1, rows_blk, 3), lambda b, i: (b, i, 0)),
        pl.BlockSpec((1, rows_blk, 3), lambda b, i: (b, i, 0)),
    ]
    if nfeat:
        args.append(grouped_f)
        in_specs.append(pl.BlockSpec((1, rows_blk, nfeat),
                                     lambda b, i: (b, i, 0)))
    for W, bias in ws:
        args.append(W)
        args.append(bias.reshape(1, -1))
        in_specs.append(pl.BlockSpec(W.shape, lambda b, i: (0, 0)))
        in_specs.append(pl.BlockSpec((1, bias.shape[0]), lambda b, i: (0, 0)))

    out = pl.pallas_call(
        functools.partial(_sa_mlp_body, nsample=nsample, blk=blk,
                          radius=radius, nfeat=nfeat),
        grid=(B, np_ // blk),
        in_specs=in_specs,
        out_specs=pl.BlockSpec((1, blk, cout), lambda b, i: (b, i, 0)),
        out_shape=jax.ShapeDtypeStruct((B, np_, cout), jnp.float32),
    )(*args)
    return out


def _fp_body(u_ref, ktr_ref, kf_ref, uf_ref, w1_ref, b1_ref, w2_ref, b2_ref,
             out_ref, *, nk):
    u = u_ref[0]
    ux = u[:, 0:1]
    uy = u[:, 1:2]
    uz = u[:, 2:3]
    KX = ktr_ref[0, 0:1, :]
    KY = ktr_ref[0, 1:2, :]
    KZ = ktr_ref[0, 2:3, :]
    a2 = (ux * ux + uy * uy) + uz * uz
    b2 = (KX * KX + KY * KY) + KZ * KZ
    dot = lax.dot_general(u, ktr_ref[0], (((1,), (0,)), ((), ())),
                          precision=lax.Precision.DEFAULT,
                          preferred_element_type=jnp.float32)
    sqd = a2 + b2 - 2.0 * dot
    jj = lax.broadcasted_iota(jnp.int32, sqd.shape, 1)

    kf = kf_ref[0]
    interp = None
    wts = []
    ds = []
    cur = sqd
    js = []
    for _ in range(3):
        m = jnp.min(cur, axis=1, keepdims=True)
        j = jnp.min(jnp.where(cur == m, jj, jnp.int32(1 << 30)),
                    axis=1, keepdims=True)
        js.append(j)
        ds.append(m)
        cur = jnp.where(jj == j, jnp.float32(3e38), cur)
    w = [1.0 / jnp.maximum(d, 1e-10) for d in ds]
    wsum = (w[0] + w[1]) + w[2]
    wn = [x / wsum for x in w]
    for i in range(3):
        oh = (jj == js[i]).astype(jnp.float32)
        r = jnp.dot(oh, kf, preferred_element_type=jnp.float32)
        interp = r * wn[i] if interp is None else interp + r * wn[i]
    cat = jnp.concatenate([interp, uf_ref[0]], axis=1)
    h = jnp.maximum(jnp.dot(cat, w1_ref[...],
                            preferred_element_type=jnp.float32)
                    + b1_ref[...], 0.0)
    out_ref[0] = jnp.maximum(jnp.dot(h, w2_ref[...],
                                     preferred_element_type=jnp.float32)
                             + b2_ref[...], 0.0)


def _fp(unknown_xyz, known_xyz, unknown_f, known_f, ws):
    B, nu, _ = unknown_xyz.shape
    nk = known_xyz.shape[1]
    ktr = jnp.moveaxis(known_xyz, -1, 1)
    (W1, b1), (W2, b2) = ws
    out = pl.pallas_call(
        functools.partial(_fp_body, nk=nk),
        grid=(B,),
        in_specs=[
            pl.BlockSpec((1, nu, 3), lambda b: (b, 0, 0)),
            pl.BlockSpec((1, 3, nk), lambda b: (b, 0, 0)),
            pl.BlockSpec((1, nk, known_f.shape[-1]), lambda b: (b, 0, 0)),
            pl.BlockSpec((1, nu, unknown_f.shape[-1]), lambda b: (b, 0, 0)),
            pl.BlockSpec(W1.shape, lambda b: (0, 0)),
            pl.BlockSpec((1, b1.shape[0]), lambda b: (0, 0)),
            pl.BlockSpec(W2.shape, lambda b: (0, 0)),
            pl.BlockSpec((1, b2.shape[0]), lambda b: (0, 0)),
        ],
        out_specs=pl.BlockSpec((1, nu, W2.shape[1]), lambda b: (b, 0, 0)),
        out_shape=jax.ShapeDtypeStruct((B, nu, W2.shape[1]), jnp.float32),
    )(unknown_xyz, ktr, known_f, unknown_f, W1, b1.reshape(1, -1),
      W2, b2.reshape(1, -1))
    return out


def _sc_gather_batched(points, idx):
    B, V, C = points.shape
    Cp = ((C + 15) // 16) * 16
    tab = points if C == Cp else jnp.pad(points, ((0, 0), (0, 0),
                                                  (0, Cp - C)))
    tab = tab.reshape(B * V, Cp)
    off = (jnp.arange(B, dtype=jnp.int32) * V).reshape(
        (B,) + (1,) * (idx.ndim - 1))
    idxg = (idx + off).reshape(-1)
    out = _sc_gather_rows(tab, idxg)
    if C != Cp:
        out = out[:, :C]
    return out.reshape(B, -1, C)


def _sa_layer(xyz, features, npoint, radius, nsample, ws, ballq_blk, mlp_blk):
    fps_inds, new_xyz = _fps(xyz, npoint)
    idx = _ball_query(new_xyz, xyz, radius, nsample, ballq_blk)
    B, np_, ns = idx.shape
    flat = idx.reshape(B, -1)
    gxyz = jnp.take_along_axis(xyz, flat[:, :, None], axis=1)
    cen_exp = jnp.repeat(new_xyz, ns, axis=1)
    gf = None
    if features is not None:
        gf = jnp.take_along_axis(features, flat[:, :, None], axis=1)
    new_f = _sa_mlp(gxyz, cen_exp, gf, ws, radius, nsample, mlp_blk)
    return new_xyz, new_f, fps_inds


def kernel(pointcloud, params):
    xyz = pointcloud[..., :3]
    sa1_xyz, sa1_f, sa1_inds = _sa_layer(
        xyz, None, 2048, 0.2, 64, params['sa1'], ballq_blk=128, mlp_blk=64)
    sa2_xyz, sa2_f, _ = _sa_layer(
        sa1_xyz, sa1_f, 1024, 0.4, 32, params['sa2'], ballq_blk=128,
        mlp_blk=128)
    sa3_xyz, sa3_f, _ = _sa_layer(
        sa2_xyz, sa2_f, 512, 0.8, 16, params['sa3'], ballq_blk=128,
        mlp_blk=256)
    sa4_xyz, sa4_f, _ = _sa_layer(
        sa3_xyz, sa3_f, 256, 1.2, 16, params['sa4'], ballq_blk=128,
        mlp_blk=256)
    f = _fp(sa3_xyz, sa4_xyz, sa3_f, sa4_f, params['fp1'])
    f = _fp(sa2_xyz, sa3_xyz, sa2_f, f, params['fp2'])
    fp2_xyz = sa2_xyz
    fp2_inds = sa1_inds[:, :fp2_xyz.shape[1]]
    return f, fp2_xyz, fp2_inds


# speedup vs baseline: 1.1543x; 1.1543x over previous
import functools
import jax
import jax.numpy as jnp
from jax import lax
from jax.experimental import pallas as pl


def _fps_body(xyz_ref, out_ref, oxyz_ref, *, n, npoint, cols):
    X = xyz_ref[0, 0]
    Y = xyz_ref[0, 1]
    Z = xyz_ref[0, 2]
    r8 = lax.broadcasted_iota(jnp.int32, (8, cols), 0)
    c8 = lax.broadcasted_iota(jnp.int32, (8, cols), 1)
    jflat = r8 * cols + c8
    real = jflat < n
    dmin0 = jnp.where(real, jnp.full((8, cols), 1e10, jnp.float32),
                      jnp.full((8, cols), -1.0, jnp.float32))

    ocols = npoint // 8
    ro = lax.broadcasted_iota(jnp.int32, (8, ocols), 0)
    co = lax.broadcasted_iota(jnp.int32, (8, ocols), 1)
    oflat = ro * ocols + co
    inds0 = jnp.zeros((8, ocols), jnp.int32)
    zf = jnp.zeros((8, ocols), jnp.float32)

    def body(i, state):
        dmin, last, inds, nx, ny, nz = state
        sel = jflat == last
        xi = jnp.sum(jnp.where(sel, X, 0.0))
        yi = jnp.sum(jnp.where(sel, Y, 0.0))
        zi = jnp.sum(jnp.where(sel, Z, 0.0))
        prev = oflat == (i - 1)
        nx = jnp.where(prev, xi, nx)
        ny = jnp.where(prev, yi, ny)
        nz = jnp.where(prev, zi, nz)
        dx = X - xi
        dy = Y - yi
        dz = Z - zi
        d = (dx * dx + dy * dy) + dz * dz
        dmin = jnp.minimum(dmin, jnp.where(real, d, -1.0))
        m = jnp.max(dmin)
        nxt = jnp.min(jnp.where(dmin == m, jflat, jnp.int32(1 << 30)))
        inds = jnp.where(oflat == i, nxt, inds)
        return dmin, nxt, inds, nx, ny, nz

    state = lax.fori_loop(1, npoint, body,
                          (dmin0, jnp.int32(0), inds0, zf, zf, zf))
    _, last, inds, nx, ny, nz = state
    sel = jflat == last
    xi = jnp.sum(jnp.where(sel, X, 0.0))
    yi = jnp.sum(jnp.where(sel, Y, 0.0))
    zi = jnp.sum(jnp.where(sel, Z, 0.0))
    fin = oflat == (npoint - 1)
    out_ref[0] = inds
    oxyz_ref[0, 0] = jnp.where(fin, xi, nx)
    oxyz_ref[0, 1] = jnp.where(fin, yi, ny)
    oxyz_ref[0, 2] = jnp.where(fin, zi, nz)


def _fps(xyz, npoint):
    B, N, _ = xyz.shape
    cols = ((N + 7) // 8 + 127) // 128 * 128
    npad = 8 * cols
    xt = jnp.moveaxis(xyz, -1, 1)
    xt = jnp.pad(xt, ((0, 0), (0, 0), (0, npad - N)))
    xt = xt.reshape(B, 3, 8, cols)
    out, oxyz = pl.pallas_call(
        functools.partial(_fps_body, n=N, npoint=npoint, cols=cols),
        grid=(B,),
        in_specs=[pl.BlockSpec((1, 3, 8, cols), lambda b: (b, 0, 0, 0))],
        out_specs=[
            pl.BlockSpec((1, 8, npoint // 8), lambda b: (b, 0, 0)),
            pl.BlockSpec((1, 3, 8, npoint // 8), lambda b: (b, 0, 0, 0)),
        ],
        out_shape=[
            jax.ShapeDtypeStruct((B, 8, npoint // 8), jnp.int32),
            jax.ShapeDtypeStruct((B, 3, 8, npoint // 8), jnp.float32),
        ],
    )(xt)
    new_xyz = jnp.moveaxis(oxyz.reshape(B, 3, npoint), 1, 2)
    return out.reshape(B, npoint), new_xyz


def _ballq_body(cen_ref, xyzt_ref, out_ref, *, n, npad, nsample, r2, blk):
    cen = cen_ref[0]
    cx = cen[:, 0:1]
    cy = cen[:, 1:2]
    cz = cen[:, 2:3]
    X = xyzt_ref[0, 0:1, :]
    Y = xyzt_ref[0, 1:2, :]
    Z = xyzt_ref[0, 2:3, :]
    a2 = (cx * cx + cy * cy) + cz * cz
    b2 = (X * X + Y * Y) + Z * Z
    dot = lax.dot_general(cen, xyzt_ref[0], (((1,), (0,)), ((), ())),
                          precision=lax.Precision.DEFAULT,
                          preferred_element_type=jnp.float32)
    sqd = a2 + b2 - 2.0 * dot
    jj = lax.broadcasted_iota(jnp.int32, (blk, npad), 1)
    mask = (sqd < r2) & (jj < n)
    score0 = jnp.where(mask, jj, jnp.int32(n))

    ki = lax.broadcasted_iota(jnp.int32, (blk, nsample), 1)
    nn = jnp.int32(n)
    out = jnp.full((blk, nsample), nn, jnp.int32)
    prev = jnp.full((blk, 1), -1, jnp.int32)
    f = jnp.zeros((blk, 1), jnp.int32)

    chunk = min(1024, npad)
    for start in range(0, npad, chunk):
        sc0 = score0[:, start:min(start + chunk, npad)]

        def kbody(k, st, sc0=sc0):
            prev, f, out, go = st

            def do(st2):
                prev, f, out = st2
                live = (sc0 > prev) & (f < nsample)
                score = jnp.where(live, sc0, nn)
                jmin = jnp.min(score, axis=1, keepdims=True)
                hit = jmin < nn
                out2 = jnp.where((ki == f) & hit, jmin, out)
                f2 = f + hit.astype(jnp.int32)
                prev2 = jnp.where(hit, jmin, prev)
                return prev2, f2, out2, jnp.max(jnp.where(hit, 1, 0))

            def skip(st2):
                prev, f, out = st2
                return prev, f, out, jnp.int32(0)

            return lax.cond(go > 0, do, skip, (prev, f, out))

        prev, f, out, _ = lax.fori_loop(0, nsample, kbody,
                                        (prev, f, out, jnp.int32(1)))
    first = out[:, 0:1]
    out_ref[0] = jnp.where(out < n, out, first)


def _ball_query(new_xyz, xyz, radius, nsample, blk):
    import numpy as np
    B, np_, _ = new_xyz.shape
    N = xyz.shape[1]
    npad = (N + 127) // 128 * 128
    xt = jnp.moveaxis(xyz, -1, 1)
    xt = jnp.pad(xt, ((0, 0), (0, 0), (0, npad - N)), constant_values=1e6)
    r2 = float(np.float32(radius * radius))
    out = pl.pallas_call(
        functools.partial(_ballq_body, n=N, npad=npad, nsample=nsample,
                          r2=r2, blk=blk),
        grid=(B, np_ // blk),
        in_specs=[
            pl.BlockSpec((1, blk, 3), lambda b, i: (b, i, 0)),
            pl.BlockSpec((1, 3, npad), lambda b, i: (b, 0, 0)),
        ],
        out_specs=pl.BlockSpec((1, blk, nsample), lambda b, i: (b, i, 0)),
        out_shape=jax.ShapeDtypeStruct((B, np_, nsample), jnp.int32),
    )(new_xyz, xt)
    return out


def _sa_mlp_body(gx_ref, cen_ref, *w_refs, nsample, blk, radius, nfeat):
    if nfeat:
        gf_ref = w_refs[0]
        w_refs = w_refs[1:]
    out_ref = w_refs[-1]
    w_refs = w_refs[:-1]
    gx = (gx_ref[0] - cen_ref[0]) / radius
    h = None
    nw = len(w_refs) // 2
    for li in range(nw):
        W = w_refs[2 * li][...]
        b = w_refs[2 * li + 1][...]
        if li == 0:
            acc = jnp.dot(gx[:, :3], W[:3, :],
                          preferred_element_type=jnp.float32)
            if nfeat:
                acc = acc + jnp.dot(gf_ref[0], W[3:3 + nfeat, :],
                                    preferred_element_type=jnp.float32)
        else:
            acc = jnp.dot(h, W, preferred_element_type=jnp.float32)
        h = jnp.maximum(acc + b, 0.0)
    rows, cout = h.shape
    h3 = h.reshape(blk, nsample, cout)
    out_ref[0] = jnp.max(h3, axis=1)


def _sa_mlp(grouped_xyz, cen_exp, grouped_f, ws, radius, nsample, blk):
    B, rows_total, _ = grouped_xyz.shape
    np_ = rows_total // nsample
    nfeat = 0 if grouped_f is None else grouped_f.shape[-1]
    cout = ws[-1][0].shape[1]
    rows_blk = blk * nsample

    args = [grouped_xyz, cen_exp]
    in_specs = [
        pl.BlockSpec((1, rows_blk, 3), lambda b, i: (b, i, 0)),
        pl.BlockSpec((1, rows_blk, 3), lambda b, i: (b, i, 0)),
    ]
    if nfeat:
        args.append(grouped_f)
        in_specs.append(pl.BlockSpec((1, rows_blk, nfeat),
                                     lambda b, i: (b, i, 0)))
    for W, bias in ws:
        args.append(W)
        args.append(bias.reshape(1, -1))
        in_specs.append(pl.BlockSpec(W.shape, lambda b, i: (0, 0)))
        in_specs.append(pl.BlockSpec((1, bias.shape[0]), lambda b, i: (0, 0)))

    out = pl.pallas_call(
        functools.partial(_sa_mlp_body, nsample=nsample, blk=blk,
                          radius=radius, nfeat=nfeat),
        grid=(B, np_ // blk),
        in_specs=in_specs,
        out_specs=pl.BlockSpec((1, blk, cout), lambda b, i: (b, i, 0)),
        out_shape=jax.ShapeDtypeStruct((B, np_, cout), jnp.float32),
    )(*args)
    return out


def _fp_body(u_ref, ktr_ref, kf_ref, uf_ref, w1_ref, b1_ref, w2_ref, b2_ref,
             out_ref, *, nk):
    u = u_ref[0]
    ux = u[:, 0:1]
    uy = u[:, 1:2]
    uz = u[:, 2:3]
    KX = ktr_ref[0, 0:1, :]
    KY = ktr_ref[0, 1:2, :]
    KZ = ktr_ref[0, 2:3, :]
    a2 = (ux * ux + uy * uy) + uz * uz
    b2 = (KX * KX + KY * KY) + KZ * KZ
    dot = lax.dot_general(u, ktr_ref[0], (((1,), (0,)), ((), ())),
                          precision=lax.Precision.DEFAULT,
                          preferred_element_type=jnp.float32)
    sqd = a2 + b2 - 2.0 * dot
    jj = lax.broadcasted_iota(jnp.int32, sqd.shape, 1)

    kf = kf_ref[0]
    interp = None
    wts = []
    ds = []
    cur = sqd
    js = []
    for _ in range(3):
        m = jnp.min(cur, axis=1, keepdims=True)
        j = jnp.min(jnp.where(cur == m, jj, jnp.int32(1 << 30)),
                    axis=1, keepdims=True)
        js.append(j)
        ds.append(m)
        cur = jnp.where(jj == j, jnp.float32(3e38), cur)
    w = [1.0 / jnp.maximum(d, 1e-10) for d in ds]
    wsum = (w[0] + w[1]) + w[2]
    wn = [x / wsum for x in w]
    for i in range(3):
        oh = (jj == js[i]).astype(jnp.float32)
        r = jnp.dot(oh, kf, preferred_element_type=jnp.float32)
        interp = r * wn[i] if interp is None else interp + r * wn[i]
    cat = jnp.concatenate([interp, uf_ref[0]], axis=1)
    h = jnp.maximum(jnp.dot(cat, w1_ref[...],
                            preferred_element_type=jnp.float32)
                    + b1_ref[...], 0.0)
    out_ref[0] = jnp.maximum(jnp.dot(h, w2_ref[...],
                                     preferred_element_type=jnp.float32)
                             + b2_ref[...], 0.0)


def _fp(unknown_xyz, known_xyz, unknown_f, known_f, ws):
    B, nu, _ = unknown_xyz.shape
    nk = known_xyz.shape[1]
    ktr = jnp.moveaxis(known_xyz, -1, 1)
    (W1, b1), (W2, b2) = ws
    out = pl.pallas_call(
        functools.partial(_fp_body, nk=nk),
        grid=(B,),
        in_specs=[
            pl.BlockSpec((1, nu, 3), lambda b: (b, 0, 0)),
            pl.BlockSpec((1, 3, nk), lambda b: (b, 0, 0)),
            pl.BlockSpec((1, nk, known_f.shape[-1]), lambda b: (b, 0, 0)),
            pl.BlockSpec((1, nu, unknown_f.shape[-1]), lambda b: (b, 0, 0)),
            pl.BlockSpec(W1.shape, lambda b: (0, 0)),
            pl.BlockSpec((1, b1.shape[0]), lambda b: (0, 0)),
            pl.BlockSpec(W2.shape, lambda b: (0, 0)),
            pl.BlockSpec((1, b2.shape[0]), lambda b: (0, 0)),
        ],
        out_specs=pl.BlockSpec((1, nu, W2.shape[1]), lambda b: (b, 0, 0)),
        out_shape=jax.ShapeDtypeStruct((B, nu, W2.shape[1]), jnp.float32),
    )(unknown_xyz, ktr, known_f, unknown_f, W1, b1.reshape(1, -1),
      W2, b2.reshape(1, -1))
    return out


def _sa_layer(xyz, features, npoint, radius, nsample, ws, ballq_blk, mlp_blk):
    fps_inds, new_xyz = _fps(xyz, npoint)
    idx = _ball_query(new_xyz, xyz, radius, nsample, ballq_blk)
    B, np_, ns = idx.shape
    flat = idx.reshape(B, -1)
    gxyz = jnp.take_along_axis(xyz, flat[:, :, None], axis=1)
    cen_exp = jnp.repeat(new_xyz, ns, axis=1)
    gf = None
    if features is not None:
        gf = jnp.take_along_axis(features, flat[:, :, None], axis=1)
    new_f = _sa_mlp(gxyz, cen_exp, gf, ws, radius, nsample, mlp_blk)
    return new_xyz, new_f, fps_inds


def kernel(pointcloud, params):
    xyz = pointcloud[..., :3]
    sa1_xyz, sa1_f, sa1_inds = _sa_layer(
        xyz, None, 2048, 0.2, 64, params['sa1'], ballq_blk=128, mlp_blk=64)
    sa2_xyz, sa2_f, _ = _sa_layer(
        sa1_xyz, sa1_f, 1024, 0.4, 32, params['sa2'], ballq_blk=128,
        mlp_blk=128)
    sa3_xyz, sa3_f, _ = _sa_layer(
        sa2_xyz, sa2_f, 512, 0.8, 16, params['sa3'], ballq_blk=128,
        mlp_blk=256)
    sa4_xyz, sa4_f, _ = _sa_layer(
        sa3_xyz, sa3_f, 256, 1.2, 16, params['sa4'], ballq_blk=128,
        mlp_blk=256)
    f = _fp(sa3_xyz, sa4_xyz, sa3_f, sa4_f, params['fp1'])
    f = _fp(sa2_xyz, sa3_xyz, sa2_f, f, params['fp2'])
    fp2_xyz = sa2_xyz
    fp2_inds = sa1_inds[:, :fp2_xyz.shape[1]]
    return f, fp2_xyz, fp2_inds


# speedup vs baseline: 1.2102x; 1.0485x over previous
import functools
import jax
import jax.numpy as jnp
from jax import lax
from jax.experimental import pallas as pl


def _fps_body(xyz_ref, out_ref, oxyz_ref, *, n, npoint, cols):
    X = xyz_ref[0, 0]
    Y = xyz_ref[0, 1]
    Z = xyz_ref[0, 2]
    r8 = lax.broadcasted_iota(jnp.int32, (8, cols), 0)
    c8 = lax.broadcasted_iota(jnp.int32, (8, cols), 1)
    jflat = r8 * cols + c8
    real = jflat < n
    dmin0 = jnp.where(real, jnp.full((8, cols), 1e10, jnp.float32),
                      jnp.full((8, cols), -1.0, jnp.float32))

    ocols = npoint // 8
    ro = lax.broadcasted_iota(jnp.int32, (8, ocols), 0)
    co = lax.broadcasted_iota(jnp.int32, (8, ocols), 1)
    oflat = ro * ocols + co
    inds0 = jnp.zeros((8, ocols), jnp.int32)
    zf = jnp.zeros((8, ocols), jnp.float32)

    def body(i, state):
        dmin, last, inds, nx, ny, nz = state
        sel = jflat == last
        xi = jnp.sum(jnp.where(sel, X, 0.0))
        yi = jnp.sum(jnp.where(sel, Y, 0.0))
        zi = jnp.sum(jnp.where(sel, Z, 0.0))
        prev = oflat == (i - 1)
        nx = jnp.where(prev, xi, nx)
        ny = jnp.where(prev, yi, ny)
        nz = jnp.where(prev, zi, nz)
        dx = X - xi
        dy = Y - yi
        dz = Z - zi
        d = (dx * dx + dy * dy) + dz * dz
        dmin = jnp.minimum(dmin, jnp.where(real, d, -1.0))
        m = jnp.max(dmin)
        nxt = jnp.min(jnp.where(dmin == m, jflat, jnp.int32(1 << 30)))
        inds = jnp.where(oflat == i, nxt, inds)
        return dmin, nxt, inds, nx, ny, nz

    state = lax.fori_loop(1, npoint, body,
                          (dmin0, jnp.int32(0), inds0, zf, zf, zf))
    _, last, inds, nx, ny, nz = state
    sel = jflat == last
    xi = jnp.sum(jnp.where(sel, X, 0.0))
    yi = jnp.sum(jnp.where(sel, Y, 0.0))
    zi = jnp.sum(jnp.where(sel, Z, 0.0))
    fin = oflat == (npoint - 1)
    out_ref[0] = inds
    oxyz_ref[0, 0] = jnp.where(fin, xi, nx)
    oxyz_ref[0, 1] = jnp.where(fin, yi, ny)
    oxyz_ref[0, 2] = jnp.where(fin, zi, nz)


def _fps(xyz, npoint):
    B, N, _ = xyz.shape
    cols = ((N + 7) // 8 + 127) // 128 * 128
    npad = 8 * cols
    xt = jnp.moveaxis(xyz, -1, 1)
    xt = jnp.pad(xt, ((0, 0), (0, 0), (0, npad - N)))
    xt = xt.reshape(B, 3, 8, cols)
    out, oxyz = pl.pallas_call(
        functools.partial(_fps_body, n=N, npoint=npoint, cols=cols),
        grid=(B,),
        in_specs=[pl.BlockSpec((1, 3, 8, cols), lambda b: (b, 0, 0, 0))],
        out_specs=[
            pl.BlockSpec((1, 8, npoint // 8), lambda b: (b, 0, 0)),
            pl.BlockSpec((1, 3, 8, npoint // 8), lambda b: (b, 0, 0, 0)),
        ],
        out_shape=[
            jax.ShapeDtypeStruct((B, 8, npoint // 8), jnp.int32),
            jax.ShapeDtypeStruct((B, 3, 8, npoint // 8), jnp.float32),
        ],
    )(xt)
    new_xyz = jnp.moveaxis(oxyz.reshape(B, 3, npoint), 1, 2)
    return out.reshape(B, npoint), new_xyz


def _ballq_body(cen_ref, xyzt_ref, out_ref, *, n, npad, nsample, r2, blk):
    cen = cen_ref[0]
    cx = cen[:, 0:1]
    cy = cen[:, 1:2]
    cz = cen[:, 2:3]
    X = xyzt_ref[0, 0:1, :]
    Y = xyzt_ref[0, 1:2, :]
    Z = xyzt_ref[0, 2:3, :]
    a2 = (cx * cx + cy * cy) + cz * cz
    b2 = (X * X + Y * Y) + Z * Z
    dot = lax.dot_general(cen, xyzt_ref[0], (((1,), (0,)), ((), ())),
                          precision=lax.Precision.DEFAULT,
                          preferred_element_type=jnp.float32)
    sqd = a2 + b2 - 2.0 * dot
    jj = lax.broadcasted_iota(jnp.int32, (blk, npad), 1)
    mask = (sqd < r2) & (jj < n)
    score0 = jnp.where(mask, jj, jnp.int32(n))

    ki = lax.broadcasted_iota(jnp.int32, (blk, nsample), 1)
    nn = jnp.int32(n)
    out = jnp.full((blk, nsample), nn, jnp.int32)
    prev = jnp.full((blk, 1), -1, jnp.int32)
    f = jnp.zeros((blk, 1), jnp.int32)

    chunk = min(2048, npad)
    for start in range(0, npad, chunk):
        sc0 = score0[:, start:min(start + chunk, npad)]

        def kbody(k, st, sc0=sc0):
            prev, f, out, go = st

            def do(st2):
                prev, f, out = st2
                live = (sc0 > prev) & (f < nsample)
                score = jnp.where(live, sc0, nn)
                jmin = jnp.min(score, axis=1, keepdims=True)
                hit = jmin < nn
                out2 = jnp.where((ki == f) & hit, jmin, out)
                f2 = f + hit.astype(jnp.int32)
                prev2 = jnp.where(hit, jmin, prev)
                return prev2, f2, out2, jnp.max(jnp.where(hit, 1, 0))

            def skip(st2):
                prev, f, out = st2
                return prev, f, out, jnp.int32(0)

            return lax.cond(go > 0, do, skip, (prev, f, out))

        prev, f, out, _ = lax.fori_loop(0, nsample, kbody,
                                        (prev, f, out, jnp.int32(1)))
    first = out[:, 0:1]
    out_ref[0] = jnp.where(out < n, out, first)


def _ball_query(new_xyz, xyz, radius, nsample, blk):
    import numpy as np
    B, np_, _ = new_xyz.shape
    N = xyz.shape[1]
    npad = (N + 127) // 128 * 128
    xt = jnp.moveaxis(xyz, -1, 1)
    xt = jnp.pad(xt, ((0, 0), (0, 0), (0, npad - N)), constant_values=1e6)
    r2 = float(np.float32(radius * radius))
    out = pl.pallas_call(
        functools.partial(_ballq_body, n=N, npad=npad, nsample=nsample,
                          r2=r2, blk=blk),
        grid=(B, np_ // blk),
        in_specs=[
            pl.BlockSpec((1, blk, 3), lambda b, i: (b, i, 0)),
            pl.BlockSpec((1, 3, npad), lambda b, i: (b, 0, 0)),
        ],
        out_specs=pl.BlockSpec((1, blk, nsample), lambda b, i: (b, i, 0)),
        out_shape=jax.ShapeDtypeStruct((B, np_, nsample), jnp.int32),
    )(new_xyz, xt)
    return out


def _sa_mlp_body(gx_ref, cen_ref, *w_refs, nsample, blk, radius, nfeat):
    if nfeat:
        gf_ref = w_refs[0]
        w_refs = w_refs[1:]
    out_ref = w_refs[-1]
    w_refs = w_refs[:-1]
    gx = (gx_ref[0] - cen_ref[0]) / radius
    h = None
    nw = len(w_refs) // 2
    for li in range(nw):
        W = w_refs[2 * li][...]
        b = w_refs[2 * li + 1][...]
        if li == 0:
            acc = jnp.dot(gx[:, :3], W[:3, :],
                          preferred_element_type=jnp.float32)
            if nfeat:
                acc = acc + jnp.dot(gf_ref[0], W[3:3 + nfeat, :],
                                    preferred_element_type=jnp.float32)
        else:
            acc = jnp.dot(h, W, preferred_element_type=jnp.float32)
        h = jnp.maximum(acc + b, 0.0)
    rows, cout = h.shape
    h3 = h.reshape(blk, nsample, cout)
    out_ref[0] = jnp.max(h3, axis=1)


def _sa_mlp(grouped_xyz, cen_exp, grouped_f, ws, radius, nsample, blk):
    B, rows_total, _ = grouped_xyz.shape
    np_ = rows_total // nsample
    nfeat = 0 if grouped_f is None else grouped_f.shape[-1]
    cout = ws[-1][0].shape[1]
    rows_blk = blk * nsample

    args = [grouped_xyz, cen_exp]
    in_specs = [
        pl.BlockSpec((1, rows_blk, 3), lambda b, i: (b, i, 0)),
        pl.BlockSpec((1, rows_blk, 3), lambda b, i: (b, i, 0)),
    ]
    if nfeat:
        args.append(grouped_f)
        in_specs.append(pl.BlockSpec((1, rows_blk, nfeat),
                                     lambda b, i: (b, i, 0)))
    for W, bias in ws:
        args.append(W)
        args.append(bias.reshape(1, -1))
        in_specs.append(pl.BlockSpec(W.shape, lambda b, i: (0, 0)))
        in_specs.append(pl.BlockSpec((1, bias.shape[0]), lambda b, i: (0, 0)))

    out = pl.pallas_call(
        functools.partial(_sa_mlp_body, nsample=nsample, blk=blk,
                          radius=radius, nfeat=nfeat),
        grid=(B, np_ // blk),
        in_specs=in_specs,
        out_specs=pl.BlockSpec((1, blk, cout), lambda b, i: (b, i, 0)),
        out_shape=jax.ShapeDtypeStruct((B, np_, cout), jnp.float32),
    )(*args)
    return out


def _fp_body(u_ref, ktr_ref, kf_ref, uf_ref, w1_ref, b1_ref, w2_ref, b2_ref,
             out_ref, *, nk):
    u = u_ref[0]
    ux = u[:, 0:1]
    uy = u[:, 1:2]
    uz = u[:, 2:3]
    KX = ktr_ref[0, 0:1, :]
    KY = ktr_ref[0, 1:2, :]
    KZ = ktr_ref[0, 2:3, :]
    a2 = (ux * ux + uy * uy) + uz * uz
    b2 = (KX * KX + KY * KY) + KZ * KZ
    dot = lax.dot_general(u, ktr_ref[0], (((1,), (0,)), ((), ())),
                          precision=lax.Precision.DEFAULT,
                          preferred_element_type=jnp.float32)
    sqd = a2 + b2 - 2.0 * dot
    jj = lax.broadcasted_iota(jnp.int32, sqd.shape, 1)

    kf = kf_ref[0]
    interp = None
    wts = []
    ds = []
    cur = sqd
    js = []
    for _ in range(3):
        m = jnp.min(cur, axis=1, keepdims=True)
        j = jnp.min(jnp.where(cur == m, jj, jnp.int32(1 << 30)),
                    axis=1, keepdims=True)
        js.append(j)
        ds.append(m)
        cur = jnp.where(jj == j, jnp.float32(3e38), cur)
    w = [1.0 / jnp.maximum(d, 1e-10) for d in ds]
    wsum = (w[0] + w[1]) + w[2]
    wn = [x / wsum for x in w]
    for i in range(3):
        oh = (jj == js[i]).astype(jnp.float32)
        r = jnp.dot(oh, kf, preferred_element_type=jnp.float32)
        interp = r * wn[i] if interp is None else interp + r * wn[i]
    cat = jnp.concatenate([interp, uf_ref[0]], axis=1)
    h = jnp.maximum(jnp.dot(cat, w1_ref[...],
                            preferred_element_type=jnp.float32)
                    + b1_ref[...], 0.0)
    out_ref[0] = jnp.maximum(jnp.dot(h, w2_ref[...],
                                     preferred_element_type=jnp.float32)
                             + b2_ref[...], 0.0)


def _fp(unknown_xyz, known_xyz, unknown_f, known_f, ws):
    B, nu, _ = unknown_xyz.shape
    nk = known_xyz.shape[1]
    ktr = jnp.moveaxis(known_xyz, -1, 1)
    (W1, b1), (W2, b2) = ws
    out = pl.pallas_call(
        functools.partial(_fp_body, nk=nk),
        grid=(B,),
        in_specs=[
            pl.BlockSpec((1, nu, 3), lambda b: (b, 0, 0)),
            pl.BlockSpec((1, 3, nk), lambda b: (b, 0, 0)),
            pl.BlockSpec((1, nk, known_f.shape[-1]), lambda b: (b, 0, 0)),
            pl.BlockSpec((1, nu, unknown_f.shape[-1]), lambda b: (b, 0, 0)),
            pl.BlockSpec(W1.shape, lambda b: (0, 0)),
            pl.BlockSpec((1, b1.shape[0]), lambda b: (0, 0)),
            pl.BlockSpec(W2.shape, lambda b: (0, 0)),
            pl.BlockSpec((1, b2.shape[0]), lambda b: (0, 0)),
        ],
        out_specs=pl.BlockSpec((1, nu, W2.shape[1]), lambda b: (b, 0, 0)),
        out_shape=jax.ShapeDtypeStruct((B, nu, W2.shape[1]), jnp.float32),
    )(unknown_xyz, ktr, known_f, unknown_f, W1, b1.reshape(1, -1),
      W2, b2.reshape(1, -1))
    return out


def _sa_layer(xyz, features, npoint, radius, nsample, ws, ballq_blk, mlp_blk):
    fps_inds, new_xyz = _fps(xyz, npoint)
    idx = _ball_query(new_xyz, xyz, radius, nsample, ballq_blk)
    B, np_, ns = idx.shape
    flat = idx.reshape(B, -1)
    gxyz = jnp.take_along_axis(xyz, flat[:, :, None], axis=1)
    cen_exp = jnp.repeat(new_xyz, ns, axis=1)
    gf = None
    if features is not None:
        gf = jnp.take_along_axis(features, flat[:, :, None], axis=1)
    new_f = _sa_mlp(gxyz, cen_exp, gf, ws, radius, nsample, mlp_blk)
    return new_xyz, new_f, fps_inds


def kernel(pointcloud, params):
    xyz = pointcloud[..., :3]
    sa1_xyz, sa1_f, sa1_inds = _sa_layer(
        xyz, None, 2048, 0.2, 64, params['sa1'], ballq_blk=128, mlp_blk=64)
    sa2_xyz, sa2_f, _ = _sa_layer(
        sa1_xyz, sa1_f, 1024, 0.4, 32, params['sa2'], ballq_blk=128,
        mlp_blk=128)
    sa3_xyz, sa3_f, _ = _sa_layer(
        sa2_xyz, sa2_f, 512, 0.8, 16, params['sa3'], ballq_blk=128,
        mlp_blk=256)
    sa4_xyz, sa4_f, _ = _sa_layer(
        sa3_xyz, sa3_f, 256, 1.2, 16, params['sa4'], ballq_blk=128,
        mlp_blk=256)
    f = _fp(sa3_xyz, sa4_xyz, sa3_f, sa4_f, params['fp1'])
    f = _fp(sa2_xyz, sa3_xyz, sa2_f, f, params['fp2'])
    fp2_xyz = sa2_xyz
    fp2_inds = sa1_inds[:, :fp2_xyz.shape[1]]
    return f, fp2_xyz, fp2_inds
